# Initial kernel scaffold; baseline (speedup 1.0000x reference)
import jax
import jax.numpy as jnp
from jax.experimental import pallas as pl


def kernel(x, weight):
    raise NotImplementedError("write your pallas kernel here")


# speedup vs baseline: 11.4071x; 11.4071x over previous
import functools

import jax
import jax.numpy as jnp
from jax.experimental import pallas as pl
from jax.experimental.pallas import tpu as pltpu


def _conv_block(w_ref, xs_ref, oh, ow):
    y = None
    for da in (0, 1):
        for db in (0, 1):
            t = xs_ref[0, :, da:da + oh, db:db + ow]
            c = jax.lax.dot_general(
                w_ref[da, db], t,
                dimension_numbers=(((1,), (0,)), ((), ())),
                preferred_element_type=jnp.float32)
            y = c if y is None else y + c
    return y


def _stats_kernel(w_ref, xs_ref, acc_ref, *, oh, ow):
    j = pl.program_id(1)

    @pl.when(j == 0)
    def _():
        acc_ref[...] = jnp.zeros_like(acc_ref)

    y = _conv_block(w_ref, xs_ref, oh, ow)
    acc_ref[0, 0] += jnp.sum(y, axis=1)
    acc_ref[0, 1] += jnp.sum(y * y, axis=1)


def _norm_kernel(w_ref, acc_ref, xs_ref, out_ref, *, m_rows, eps, slope,
                 oh, ow):
    oc = out_ref.shape[1]
    tot = acc_ref[0] + acc_ref[1]
    s = jnp.sum(tot[0], axis=1, keepdims=True)
    ss = jnp.sum(tot[1], axis=1, keepdims=True)
    mean = s * (1.0 / m_rows)
    denom_n = max(m_rows - 1.0, 1.0)
    var = jnp.maximum(ss - m_rows * mean * mean, 0.0) * (1.0 / denom_n)
    scale = 1.0 / (jnp.sqrt(var) + eps)
    mean_b = jnp.broadcast_to(mean, (oc, ow)).reshape(oc, 1, ow)
    scale_b = jnp.broadcast_to(scale, (oc, ow)).reshape(oc, 1, ow)

    y = _conv_block(w_ref, xs_ref, oh, ow)
    h = (y - mean_b) * scale_b
    out_ref[0] = jnp.maximum(h, slope * h)


def _feature_layer(x, weight, *, negative_slope=0.2, eps=1e-6):
    N, C, H, W = x.shape
    OC, Cw, KH, KW = weight.shape
    assert Cw == C and KH == 4 and KW == 4
    OH, OW = H // 2, W // 2
    M = N * OH * OW

    xp = jnp.pad(x, ((0, 0), (0, 0), (1, 1), (1, 1)))
    xs = xp.reshape(N, C, OH + 1, 2, OW + 1, 2)
    xs = xs.transpose(0, 1, 3, 5, 2, 4).reshape(N, C * 4, OH + 1, OW + 1)

    w4 = weight.reshape(OC, C, 2, 2, 2, 2).transpose(2, 4, 0, 1, 3, 5)
    w4 = w4.reshape(2, 2, OC, C * 4).astype(jnp.float32)

    n_half = N // 2

    stats_kernel = functools.partial(_stats_kernel, oh=OH, ow=OW)
    acc = pl.pallas_call(
        stats_kernel,
        out_shape=jax.ShapeDtypeStruct((2, 2, OC, OW), jnp.float32),
        grid=(2, n_half),
        in_specs=[
            pl.BlockSpec((2, 2, OC, C * 4), lambda c, j: (0, 0, 0, 0)),
            pl.BlockSpec((1, C * 4, OH + 1, OW + 1),
                         lambda c, j: (c * n_half + j, 0, 0, 0)),
        ],
        out_specs=pl.BlockSpec((1, 2, OC, OW), lambda c, j: (c, 0, 0, 0)),
        compiler_params=pltpu.CompilerParams(
            dimension_semantics=("parallel", "arbitrary"),
            vmem_limit_bytes=64 * 1024 * 1024,
        ),
    )(w4, xs)

    norm_kernel = functools.partial(
        _norm_kernel, m_rows=float(M), eps=eps, slope=negative_slope,
        oh=OH, ow=OW)
    out = pl.pallas_call(
        norm_kernel,
        out_shape=jax.ShapeDtypeStruct((N, OC, OH, OW), jnp.float32),
        grid=(N,),
        in_specs=[
            pl.BlockSpec((2, 2, OC, C * 4), lambda i: (0, 0, 0, 0)),
            pl.BlockSpec((2, 2, OC, OW), lambda i: (0, 0, 0, 0)),
            pl.BlockSpec((1, C * 4, OH + 1, OW + 1), lambda i: (i, 0, 0, 0)),
        ],
        out_specs=pl.BlockSpec((1, OC, OH, OW), lambda i: (i, 0, 0, 0)),
        compiler_params=pltpu.CompilerParams(
            dimension_semantics=("parallel",),
            vmem_limit_bytes=64 * 1024 * 1024,
        ),
    )(w4, acc, xs)

    return out


def kernel(x, weight):
    return _feature_layer(x, weight)


# speedup vs baseline: 13.3510x; 1.1704x over previous
import functools

import jax
import jax.numpy as jnp
from jax.experimental import pallas as pl
from jax.experimental.pallas import tpu as pltpu


def _conv_flat(w_ref, xs_ref, e_ref, spread_ref, m_out, ow):
    taps = [xs_ref[0, :, off:off + m_out] for off in (0, 1, ow, ow + 1)]
    corr = jax.lax.dot_general(
        e_ref[0], spread_ref[...],
        dimension_numbers=(((1,), (0,)), ((), ())),
        preferred_element_type=jnp.float32).astype(jnp.bfloat16)
    rhs = jnp.concatenate(taps + [corr], axis=0)
    y = jax.lax.dot_general(
        w_ref[...], rhs,
        dimension_numbers=(((1,), (0,)), ((), ())),
        preferred_element_type=jnp.float32)
    return y


def _stats_kernel(w_ref, xs_ref, e_ref, spread_ref, acc_ref, *, m_out, ow):
    j = pl.program_id(1)

    @pl.when(j == 0)
    def _():
        acc_ref[...] = jnp.zeros_like(acc_ref)

    y = _conv_flat(w_ref, xs_ref, e_ref, spread_ref, m_out, ow)
    oc = acc_ref.shape[2]
    lanes = acc_ref.shape[3]
    s = jnp.sum(y, axis=1, keepdims=True)
    ss = jnp.sum(y * y, axis=1, keepdims=True)
    acc_ref[0, 0] += jnp.broadcast_to(s, (oc, lanes))
    acc_ref[0, 1] += jnp.broadcast_to(ss, (oc, lanes))


def _norm_kernel(w_ref, acc_ref, xs_ref, e_ref, spread_ref, out_ref, *,
                 m_rows, eps, slope, m_out, ow):
    tot = acc_ref[0] + acc_ref[1]
    s = tot[0, :, :1]
    ss = tot[1, :, :1]
    mean = s * (1.0 / m_rows)
    denom_n = max(m_rows - 1.0, 1.0)
    var = jnp.maximum(ss - m_rows * mean * mean, 0.0) * (1.0 / denom_n)
    scale = 1.0 / (jnp.sqrt(var) + eps)

    y = _conv_flat(w_ref, xs_ref, e_ref, spread_ref, m_out, ow)
    h = (y - mean) * scale
    out_ref[0] = jnp.maximum(h, slope * h)


def _feature_layer(x, weight, *, negative_slope=0.2, eps=1e-6):
    N, C, H, W = x.shape
    OC, Cw, KH, KW = weight.shape
    assert Cw == C and KH == 4 and KW == 4
    OH, OW = H // 2, W // 2
    M = N * OH * OW
    m_out = OH * OW
    lpad = ((m_out + OW + 129) + 127) // 128 * 128

    f32 = jnp.float32
    bf16 = jnp.bfloat16

    xp = jnp.pad(x, ((0, 0), (0, 0), (1, 1), (1, 1)))
    s2d = xp.reshape(N, C, OH + 1, 2, OW + 1, 2)
    s2d = s2d.transpose(0, 1, 3, 5, 2, 4)
    xs = s2d[..., :OW].reshape(N, C * 4, (OH + 1) * OW)
    xs = jnp.pad(xs, ((0, 0), (0, 0), (0, lpad - (OH + 1) * OW)))
    xs = xs.astype(bf16)

    A = s2d[:, :, :, 0, :, W // 2].reshape(N, C * 2, OH + 1)
    B = s2d[:, :, :, 1, :, 0].reshape(N, C * 2, OH + 1)
    Bp = jnp.pad(B, ((0, 0), (0, 0), (0, 1)))
    E = jnp.concatenate([
        A[:, :, :OH], A[:, :, 1:OH + 1],
        Bp[:, :, 1:OH + 1], Bp[:, :, 2:OH + 2],
    ], axis=1).astype(bf16)

    wr = weight.reshape(OC, C, 2, 2, 2, 2).transpose(2, 4, 0, 1, 3, 5)
    w_main = wr.reshape(4, OC, C * 4)
    w_main = jnp.concatenate([w_main[i] for i in range(4)], axis=1)
    wk = weight.reshape(OC, C, 2, 2, 4)
    wA = wk[..., 2].transpose(0, 2, 1, 3).reshape(OC, 4 * C)
    wB = (-wk[..., 3]).transpose(0, 2, 1, 3).reshape(OC, 4 * C)
    w_all = jnp.concatenate([w_main, wA, wB], axis=1).astype(bf16)

    oh_idx = jax.lax.broadcasted_iota(jnp.int32, (OH, m_out), 0)
    m_idx = jax.lax.broadcasted_iota(jnp.int32, (OH, m_out), 1)
    spread = (m_idx == oh_idx * OW + (OW - 1)).astype(bf16)

    n_half = N // 2
    cparams = pltpu.CompilerParams(
        dimension_semantics=("parallel", "arbitrary"),
        vmem_limit_bytes=64 * 1024 * 1024,
    )

    stats_kernel = functools.partial(_stats_kernel, m_out=m_out, ow=OW)
    acc = pl.pallas_call(
        stats_kernel,
        out_shape=jax.ShapeDtypeStruct((2, 2, OC, 128), f32),
        grid=(2, n_half),
        in_specs=[
            pl.BlockSpec((OC, 96), lambda c, j: (0, 0)),
            pl.BlockSpec((1, C * 4, lpad), lambda c, j: (c * n_half + j, 0, 0)),
            pl.BlockSpec((1, 32, OH), lambda c, j: (c * n_half + j, 0, 0)),
            pl.BlockSpec((OH, m_out), lambda c, j: (0, 0)),
        ],
        out_specs=pl.BlockSpec((1, 2, OC, 128), lambda c, j: (c, 0, 0, 0)),
        compiler_params=cparams,
    )(w_all, xs, E, spread)

    norm_kernel = functools.partial(
        _norm_kernel, m_rows=float(M), eps=eps, slope=negative_slope,
        m_out=m_out, ow=OW)
    out = pl.pallas_call(
        norm_kernel,
        out_shape=jax.ShapeDtypeStruct((N, OC, m_out), f32),
        grid=(2, n_half),
        in_specs=[
            pl.BlockSpec((OC, 96), lambda c, j: (0, 0)),
            pl.BlockSpec((2, 2, OC, 128), lambda c, j: (0, 0, 0, 0)),
            pl.BlockSpec((1, C * 4, lpad), lambda c, j: (c * n_half + j, 0, 0)),
            pl.BlockSpec((1, 32, OH), lambda c, j: (c * n_half + j, 0, 0)),
            pl.BlockSpec((OH, m_out), lambda c, j: (0, 0)),
        ],
        out_specs=pl.BlockSpec((1, OC, m_out),
                               lambda c, j: (c * n_half + j, 0, 0)),
        compiler_params=cparams,
    )(w_all, acc, xs, E, spread)

    return out.reshape(N, OC, OH, OW)


def kernel(x, weight):
    return _feature_layer(x, weight)


# speedup vs baseline: 22.8970x; 1.7150x over previous
import functools

import jax
import jax.numpy as jnp
from jax.experimental import pallas as pl
from jax.experimental.pallas import tpu as pltpu


def _round_up(a, b):
    return (a + b - 1) // b * b


def _prep_kernel(x_ref, r_ref, l_ref, xs_ref, *, c_in, row_blk, j_rows, ow):
    for c in range(c_in):
        xb = x_ref[0, c].astype(jnp.bfloat16)
        t = jax.lax.dot_general(
            xb, r_ref[...],
            dimension_numbers=(((1,), (0,)), ((), ())),
            preferred_element_type=jnp.float32).astype(jnp.bfloat16)
        y = jax.lax.dot_general(
            l_ref[...], t,
            dimension_numbers=(((1,), (0,)), ((), ())),
            preferred_element_type=jnp.float32).astype(jnp.bfloat16)
        for p in range(2):
            for q in range(2):
                xs_ref[0, c * 4 + p * 2 + q] = (
                    y[p * row_blk:p * row_blk + j_rows, q * ow:(q + 1) * ow])


def _conv_flat(w_ref, xs_ref, e_ref, spread_ref, m_out, ow):
    taps = [xs_ref[0, :, off:off + m_out] for off in (0, 1, ow, ow + 1)]
    corr = jax.lax.dot_general(
        e_ref[0], spread_ref[...],
        dimension_numbers=(((1,), (0,)), ((), ())),
        preferred_element_type=jnp.float32).astype(jnp.bfloat16)
    rhs = jnp.concatenate(taps + [corr], axis=0)
    y = jax.lax.dot_general(
        w_ref[...], rhs,
        dimension_numbers=(((1,), (0,)), ((), ())),
        preferred_element_type=jnp.float32)
    return y


def _stats_kernel(w_ref, xs_ref, e_ref, spread_ref, acc_ref, *, m_out, ow):
    j = pl.program_id(1)

    @pl.when(j == 0)
    def _():
        acc_ref[...] = jnp.zeros_like(acc_ref)

    y = _conv_flat(w_ref, xs_ref, e_ref, spread_ref, m_out, ow)
    oc = acc_ref.shape[2]
    lanes = acc_ref.shape[3]
    s = jnp.sum(y, axis=1, keepdims=True)
    ss = jnp.sum(y * y, axis=1, keepdims=True)
    acc_ref[0, 0] += jnp.broadcast_to(s, (oc, lanes))
    acc_ref[0, 1] += jnp.broadcast_to(ss, (oc, lanes))


def _norm_kernel(w_ref, acc_ref, xs_ref, e_ref, spread_ref, out_ref, *,
                 m_rows, eps, slope, m_out, ow):
    tot = acc_ref[0] + acc_ref[1]
    s = tot[0, :, :1]
    ss = tot[1, :, :1]
    mean = s * (1.0 / m_rows)
    denom_n = max(m_rows - 1.0, 1.0)
    var = jnp.maximum(ss - m_rows * mean * mean, 0.0) * (1.0 / denom_n)
    scale = 1.0 / (jnp.sqrt(var) + eps)

    y = _conv_flat(w_ref, xs_ref, e_ref, spread_ref, m_out, ow)
    h = (y - mean) * scale
    out_ref[0] = jnp.maximum(h, slope * h)


def _feature_layer(x, weight, *, negative_slope=0.2, eps=1e-6):
    N, C, H, W = x.shape
    OC, Cw, KH, KW = weight.shape
    assert Cw == C and KH == 4 and KW == 4
    OH, OW = H // 2, W // 2
    M = N * OH * OW
    m_out = OH * OW
    j_rows = OH + 2
    row_blk = _round_up(j_rows, 16)
    lpad = j_rows * OW

    f32 = jnp.float32
    bf16 = jnp.bfloat16

    w_idx = jax.lax.broadcasted_iota(jnp.int32, (W, 2 * OW), 0)
    col = jax.lax.broadcasted_iota(jnp.int32, (W, 2 * OW), 1)
    rsel = (w_idx == 2 * (col % OW) + (col // OW) - 1).astype(bf16)
    row = jax.lax.broadcasted_iota(jnp.int32, (2 * row_blk, H), 0)
    r_idx = jax.lax.broadcasted_iota(jnp.int32, (2 * row_blk, H), 1)
    lsel = (r_idx == 2 * (row % row_blk) + (row // row_blk) - 1).astype(bf16)

    def phase_rows(col2d):
        colp = jnp.pad(col2d, ((0, 0), (0, 0), (1, 1)))
        ph = colp.reshape(N, C, H // 2 + 1, 2).transpose(0, 1, 3, 2)
        return ph.reshape(N, C * 2, H // 2 + 1)

    A = phase_rows(x[:, :, :, W - 1])
    B = phase_rows(x[:, :, :, 0])
    Bp = jnp.pad(B, ((0, 0), (0, 0), (0, 1)))
    E = jnp.concatenate([
        A[:, :, :OH], A[:, :, 1:OH + 1],
        Bp[:, :, 1:OH + 1], Bp[:, :, 2:OH + 2],
    ], axis=1).astype(bf16)

    wr = weight.reshape(OC, C, 2, 2, 2, 2).transpose(2, 4, 0, 1, 3, 5)
    w_main = wr.reshape(4, OC, C * 4)
    w_main = jnp.concatenate([w_main[i] for i in range(4)], axis=1)
    wk = weight.reshape(OC, C, 2, 2, 4)
    wA = wk[..., 2].transpose(0, 2, 1, 3).reshape(OC, 4 * C)
    wB = (-wk[..., 3]).transpose(0, 2, 1, 3).reshape(OC, 4 * C)
    w_all = jnp.concatenate([w_main, wA, wB], axis=1).astype(bf16)

    oh_idx = jax.lax.broadcasted_iota(jnp.int32, (OH, m_out), 0)
    m_idx = jax.lax.broadcasted_iota(jnp.int32, (OH, m_out), 1)
    spread = (m_idx == oh_idx * OW + (OW - 1)).astype(bf16)

    n_half = N // 2
    cparams = pltpu.CompilerParams(
        dimension_semantics=("parallel", "arbitrary"),
        vmem_limit_bytes=64 * 1024 * 1024,
    )

    prep_kernel = functools.partial(
        _prep_kernel, c_in=C, row_blk=row_blk, j_rows=j_rows, ow=OW)
    xs4 = pl.pallas_call(
        prep_kernel,
        out_shape=jax.ShapeDtypeStruct((N, C * 4, j_rows, OW), bf16),
        grid=(2, n_half),
        in_specs=[
            pl.BlockSpec((1, C, H, W), lambda c, j: (c * n_half + j, 0, 0, 0)),
            pl.BlockSpec((W, 2 * OW), lambda c, j: (0, 0)),
            pl.BlockSpec((2 * row_blk, H), lambda c, j: (0, 0)),
        ],
        out_specs=pl.BlockSpec((1, C * 4, j_rows, OW),
                               lambda c, j: (c * n_half + j, 0, 0, 0)),
        compiler_params=cparams,
    )(x, rsel, lsel)
    xs = xs4.reshape(N, C * 4, lpad)

    stats_kernel = functools.partial(_stats_kernel, m_out=m_out, ow=OW)
    acc = pl.pallas_call(
        stats_kernel,
        out_shape=jax.ShapeDtypeStruct((2, 2, OC, 128), f32),
        grid=(2, n_half),
        in_specs=[
            pl.BlockSpec((OC, 96), lambda c, j: (0, 0)),
            pl.BlockSpec((1, C * 4, lpad), lambda c, j: (c * n_half + j, 0, 0)),
            pl.BlockSpec((1, 32, OH), lambda c, j: (c * n_half + j, 0, 0)),
            pl.BlockSpec((OH, m_out), lambda c, j: (0, 0)),
        ],
        out_specs=pl.BlockSpec((1, 2, OC, 128), lambda c, j: (c, 0, 0, 0)),
        compiler_params=cparams,
    )(w_all, xs, E, spread)

    norm_kernel = functools.partial(
        _norm_kernel, m_rows=float(M), eps=eps, slope=negative_slope,
        m_out=m_out, ow=OW)
    out = pl.pallas_call(
        norm_kernel,
        out_shape=jax.ShapeDtypeStruct((N, OC, m_out), f32),
        grid=(2, n_half),
        in_specs=[
            pl.BlockSpec((OC, 96), lambda c, j: (0, 0)),
            pl.BlockSpec((2, 2, OC, 128), lambda c, j: (0, 0, 0, 0)),
            pl.BlockSpec((1, C * 4, lpad), lambda c, j: (c * n_half + j, 0, 0)),
            pl.BlockSpec((1, 32, OH), lambda c, j: (c * n_half + j, 0, 0)),
            pl.BlockSpec((OH, m_out), lambda c, j: (0, 0)),
        ],
        out_specs=pl.BlockSpec((1, OC, m_out),
                               lambda c, j: (c * n_half + j, 0, 0)),
        compiler_params=cparams,
    )(w_all, acc, xs, E, spread)

    return out.reshape(N, OC, OH, OW)


def kernel(x, weight):
    return _feature_layer(x, weight)


# speedup vs baseline: 23.2382x; 1.0149x over previous
import functools

import jax
import jax.numpy as jnp
import numpy as np
from jax.experimental import pallas as pl
from jax.experimental.pallas import tpu as pltpu


def _round_up(a, b):
    return (a + b - 1) // b * b


def _prep_kernel(x_ref, r_ref, l_ref, xs_ref, *, c_in, row_blk, j_rows, ow):
    for c in range(c_in):
        xb = x_ref[0, c].astype(jnp.bfloat16)
        t = jax.lax.dot_general(
            xb, r_ref[...],
            dimension_numbers=(((1,), (0,)), ((), ())),
            preferred_element_type=jnp.float32).astype(jnp.bfloat16)
        y = jax.lax.dot_general(
            l_ref[...], t,
            dimension_numbers=(((1,), (0,)), ((), ())),
            preferred_element_type=jnp.float32).astype(jnp.bfloat16)
        for p in range(2):
            for q in range(2):
                xs_ref[0, c * 4 + p * 2 + q] = (
                    y[p * row_blk:p * row_blk + j_rows, q * ow:(q + 1) * ow])


def _conv_flat(w_ref, xs_ref, e_ref, spread_ref, m_out, ow):
    taps = [xs_ref[0, :, off:off + m_out] for off in (0, 1, ow, ow + 1)]
    corr = jax.lax.dot_general(
        e_ref[0], spread_ref[...],
        dimension_numbers=(((1,), (0,)), ((), ())),
        preferred_element_type=jnp.float32).astype(jnp.bfloat16)
    rhs = jnp.concatenate(taps + [corr], axis=0)
    y = jax.lax.dot_general(
        w_ref[...], rhs,
        dimension_numbers=(((1,), (0,)), ((), ())),
        preferred_element_type=jnp.float32)
    return y


def _stats_kernel(w_ref, xs_ref, e_ref, spread_ref, acc_ref, *, m_out, ow):
    j = pl.program_id(1)

    @pl.when(j == 0)
    def _():
        acc_ref[...] = jnp.zeros_like(acc_ref)

    y = _conv_flat(w_ref, xs_ref, e_ref, spread_ref, m_out, ow)
    oc = acc_ref.shape[2]
    lanes = acc_ref.shape[3]
    s = jnp.sum(y, axis=1, keepdims=True)
    ss = jnp.sum(y * y, axis=1, keepdims=True)
    acc_ref[0, 0] += jnp.broadcast_to(s, (oc, lanes))
    acc_ref[0, 1] += jnp.broadcast_to(ss, (oc, lanes))


def _norm_kernel(w_ref, acc_ref, xs_ref, e_ref, spread_ref, out_ref, *,
                 m_rows, eps, slope, m_out, ow):
    tot = acc_ref[0] + acc_ref[1]
    s = tot[0, :, :1]
    ss = tot[1, :, :1]
    mean = s * (1.0 / m_rows)
    denom_n = max(m_rows - 1.0, 1.0)
    var = jnp.maximum(ss - m_rows * mean * mean, 0.0) * (1.0 / denom_n)
    scale = 1.0 / (jnp.sqrt(var) + eps)

    y = _conv_flat(w_ref, xs_ref, e_ref, spread_ref, m_out, ow)
    h = (y - mean) * scale
    out_ref[0] = jnp.maximum(h, slope * h)


def _feature_layer(x, weight, *, negative_slope=0.2, eps=1e-6):
    N, C, H, W = x.shape
    OC, Cw, KH, KW = weight.shape
    assert Cw == C and KH == 4 and KW == 4
    OH, OW = H // 2, W // 2
    M = N * OH * OW
    m_out = OH * OW
    j_rows = OH + 2
    row_blk = _round_up(j_rows, 16)
    lpad = j_rows * OW

    f32 = jnp.float32
    bf16 = jnp.bfloat16

    rsel_np = np.zeros((W, 2 * OW), np.float32)
    for q in range(2):
        for i in range(OW):
            w_src = 2 * i + q - 1
            if 0 <= w_src < W:
                rsel_np[w_src, q * OW + i] = 1.0
    rsel = jnp.asarray(rsel_np).astype(bf16)
    lsel_np = np.zeros((2 * row_blk, H), np.float32)
    for p in range(2):
        for j in range(j_rows):
            r_src = 2 * j + p - 1
            if 0 <= r_src < H:
                lsel_np[p * row_blk + j, r_src] = 1.0
    lsel = jnp.asarray(lsel_np).astype(bf16)

    def phase_rows(col2d):
        colp = jnp.pad(col2d, ((0, 0), (0, 0), (1, 1)))
        ph = colp.reshape(N, C, H // 2 + 1, 2).transpose(0, 1, 3, 2)
        return ph.reshape(N, C * 2, H // 2 + 1)

    A = phase_rows(x[:, :, :, W - 1])
    B = phase_rows(x[:, :, :, 0])
    Bp = jnp.pad(B, ((0, 0), (0, 0), (0, 1)))
    E = jnp.concatenate([
        A[:, :, :OH], A[:, :, 1:OH + 1],
        Bp[:, :, 1:OH + 1], Bp[:, :, 2:OH + 2],
    ], axis=1).astype(bf16)

    wr = weight.reshape(OC, C, 2, 2, 2, 2).transpose(2, 4, 0, 1, 3, 5)
    w_main = wr.reshape(4, OC, C * 4)
    w_main = jnp.concatenate([w_main[i] for i in range(4)], axis=1)
    wk = weight.reshape(OC, C, 2, 2, 4)
    wA = wk[..., 2].transpose(0, 2, 1, 3).reshape(OC, 4 * C)
    wB = (-wk[..., 3]).transpose(0, 2, 1, 3).reshape(OC, 4 * C)
    w_all = jnp.concatenate([w_main, wA, wB], axis=1).astype(bf16)

    spread_np = np.zeros((OH, m_out), np.float32)
    spread_np[np.arange(OH), np.arange(OH) * OW + (OW - 1)] = 1.0
    spread = jnp.asarray(spread_np).astype(bf16)

    n_half = N // 2
    cparams = pltpu.CompilerParams(
        dimension_semantics=("parallel", "arbitrary"),
        vmem_limit_bytes=64 * 1024 * 1024,
    )

    prep_kernel = functools.partial(
        _prep_kernel, c_in=C, row_blk=row_blk, j_rows=j_rows, ow=OW)
    xs4 = pl.pallas_call(
        prep_kernel,
        out_shape=jax.ShapeDtypeStruct((N, C * 4, j_rows, OW), bf16),
        grid=(2, n_half),
        in_specs=[
            pl.BlockSpec((1, C, H, W), lambda c, j: (c * n_half + j, 0, 0, 0)),
            pl.BlockSpec((W, 2 * OW), lambda c, j: (0, 0)),
            pl.BlockSpec((2 * row_blk, H), lambda c, j: (0, 0)),
        ],
        out_specs=pl.BlockSpec((1, C * 4, j_rows, OW),
                               lambda c, j: (c * n_half + j, 0, 0, 0)),
        compiler_params=cparams,
    )(x, rsel, lsel)
    xs = xs4.reshape(N, C * 4, lpad)

    stats_kernel = functools.partial(_stats_kernel, m_out=m_out, ow=OW)
    acc = pl.pallas_call(
        stats_kernel,
        out_shape=jax.ShapeDtypeStruct((2, 2, OC, 128), f32),
        grid=(2, n_half),
        in_specs=[
            pl.BlockSpec((OC, 96), lambda c, j: (0, 0)),
            pl.BlockSpec((1, C * 4, lpad), lambda c, j: (c * n_half + j, 0, 0)),
            pl.BlockSpec((1, 32, OH), lambda c, j: (c * n_half + j, 0, 0)),
            pl.BlockSpec((OH, m_out), lambda c, j: (0, 0)),
        ],
        out_specs=pl.BlockSpec((1, 2, OC, 128), lambda c, j: (c, 0, 0, 0)),
        compiler_params=cparams,
    )(w_all, xs, E, spread)

    norm_kernel = functools.partial(
        _norm_kernel, m_rows=float(M), eps=eps, slope=negative_slope,
        m_out=m_out, ow=OW)
    out = pl.pallas_call(
        norm_kernel,
        out_shape=jax.ShapeDtypeStruct((N, OC, m_out), f32),
        grid=(2, n_half),
        in_specs=[
            pl.BlockSpec((OC, 96), lambda c, j: (0, 0)),
            pl.BlockSpec((2, 2, OC, 128), lambda c, j: (0, 0, 0, 0)),
            pl.BlockSpec((1, C * 4, lpad), lambda c, j: (c * n_half + j, 0, 0)),
            pl.BlockSpec((1, 32, OH), lambda c, j: (c * n_half + j, 0, 0)),
            pl.BlockSpec((OH, m_out), lambda c, j: (0, 0)),
        ],
        out_specs=pl.BlockSpec((1, OC, m_out),
                               lambda c, j: (c * n_half + j, 0, 0)),
        compiler_params=cparams,
    )(w_all, acc, xs, E, spread)

    return out.reshape(N, OC, OH, OW)


def kernel(x, weight):
    return _feature_layer(x, weight)


# speedup vs baseline: 24.3405x; 1.0474x over previous
import functools

import jax
import jax.numpy as jnp
import numpy as np
from jax.experimental import pallas as pl
from jax.experimental.pallas import tpu as pltpu


def _round_up(a, b):
    return (a + b - 1) // b * b


def _prep_kernel(x_ref, r_ref, l_ref, xs_ref, ecol_ref, *, c_in, row_blk,
                 j_rows, ow):
    for c in range(c_in):
        xb = x_ref[0, c].astype(jnp.bfloat16)
        t = jax.lax.dot_general(
            xb, r_ref[...],
            dimension_numbers=(((1,), (0,)), ((), ())),
            preferred_element_type=jnp.float32).astype(jnp.bfloat16)
        y = jax.lax.dot_general(
            l_ref[...], t,
            dimension_numbers=(((1,), (0,)), ((), ())),
            preferred_element_type=jnp.float32).astype(jnp.bfloat16)
        for p in range(2):
            rows = y[p * row_blk:p * row_blk + j_rows]
            for q in range(2):
                xs_ref[0, c * 4 + p * 2 + q] = rows[:, q * ow:(q + 1) * ow]
            ecol_ref[0, c * 2 + p] = jnp.concatenate(
                [rows[:, 2 * ow:2 * ow + 1], rows[:, ow:ow + 1]], axis=1)


def _conv_flat(w_ref, xs_ref, e_ref, spread_ref, m_out, ow):
    taps = [xs_ref[0, :, off:off + m_out] for off in (0, 1, ow, ow + 1)]
    corr = jax.lax.dot_general(
        e_ref[0], spread_ref[...],
        dimension_numbers=(((1,), (0,)), ((), ())),
        preferred_element_type=jnp.float32).astype(jnp.bfloat16)
    rhs = jnp.concatenate(taps + [corr], axis=0)
    y = jax.lax.dot_general(
        w_ref[...], rhs,
        dimension_numbers=(((1,), (0,)), ((), ())),
        preferred_element_type=jnp.float32)
    return y


def _stats_kernel(w_ref, xs_ref, e_ref, spread_ref, acc_ref, *, m_out, ow):
    j = pl.program_id(1)

    @pl.when(j == 0)
    def _():
        acc_ref[...] = jnp.zeros_like(acc_ref)

    y = _conv_flat(w_ref, xs_ref, e_ref, spread_ref, m_out, ow)
    oc = acc_ref.shape[2]
    lanes = acc_ref.shape[3]
    s = jnp.sum(y, axis=1, keepdims=True)
    ss = jnp.sum(y * y, axis=1, keepdims=True)
    acc_ref[0, 0] += jnp.broadcast_to(s, (oc, lanes))
    acc_ref[0, 1] += jnp.broadcast_to(ss, (oc, lanes))


def _norm_kernel(w_ref, acc_ref, xs_ref, e_ref, spread_ref, out_ref, *,
                 m_rows, eps, slope, m_out, ow):
    tot = acc_ref[0] + acc_ref[1]
    s = tot[0, :, :1]
    ss = tot[1, :, :1]
    mean = s * (1.0 / m_rows)
    denom_n = max(m_rows - 1.0, 1.0)
    var = jnp.maximum(ss - m_rows * mean * mean, 0.0) * (1.0 / denom_n)
    scale = 1.0 / (jnp.sqrt(var) + eps)

    y = _conv_flat(w_ref, xs_ref, e_ref, spread_ref, m_out, ow)
    h = (y - mean) * scale
    out_ref[0] = jnp.maximum(h, slope * h)


def _feature_layer(x, weight, *, negative_slope=0.2, eps=1e-6):
    N, C, H, W = x.shape
    OC, Cw, KH, KW = weight.shape
    assert Cw == C and KH == 4 and KW == 4
    OH, OW = H // 2, W // 2
    M = N * OH * OW
    m_out = OH * OW
    j_rows = OH + 2
    row_blk = _round_up(j_rows, 16)
    lpad = j_rows * OW

    f32 = jnp.float32
    bf16 = jnp.bfloat16

    rsel_np = np.zeros((W, 2 * OW + 2), np.float32)
    for q in range(2):
        for i in range(OW):
            w_src = 2 * i + q - 1
            if 0 <= w_src < W:
                rsel_np[w_src, q * OW + i] = 1.0
    rsel_np[W - 1, 2 * OW] = 1.0
    rsel = jnp.asarray(rsel_np).astype(bf16)
    lsel_np = np.zeros((2 * row_blk, H), np.float32)
    for p in range(2):
        for j in range(j_rows):
            r_src = 2 * j + p - 1
            if 0 <= r_src < H:
                lsel_np[p * row_blk + j, r_src] = 1.0
    lsel = jnp.asarray(lsel_np).astype(bf16)

    n_half = N // 2
    cparams = pltpu.CompilerParams(
        dimension_semantics=("parallel", "arbitrary"),
        vmem_limit_bytes=64 * 1024 * 1024,
    )

    wr = weight.reshape(OC, C, 2, 2, 2, 2).transpose(2, 4, 0, 1, 3, 5)
    w_main = wr.reshape(4, OC, C * 4)
    w_main = jnp.concatenate([w_main[i] for i in range(4)], axis=1)
    wk = weight.reshape(OC, C, 2, 2, 4)
    wA = wk[..., 2].transpose(0, 2, 1, 3).reshape(OC, 4 * C)
    wB = (-wk[..., 3]).transpose(0, 2, 1, 3).reshape(OC, 4 * C)
    w_all = jnp.concatenate([w_main, wA, wB], axis=1).astype(bf16)

    spread_np = np.zeros((OH, m_out), np.float32)
    spread_np[np.arange(OH), np.arange(OH) * OW + (OW - 1)] = 1.0
    spread = jnp.asarray(spread_np).astype(bf16)

    prep_kernel = functools.partial(
        _prep_kernel, c_in=C, row_blk=row_blk, j_rows=j_rows, ow=OW)
    xs4, ecol = pl.pallas_call(
        prep_kernel,
        out_shape=(jax.ShapeDtypeStruct((N, C * 4, j_rows, OW), bf16),
                   jax.ShapeDtypeStruct((N, C * 2, j_rows, 2), bf16)),
        grid=(2, n_half),
        in_specs=[
            pl.BlockSpec((1, C, H, W), lambda c, j: (c * n_half + j, 0, 0, 0)),
            pl.BlockSpec((W, 2 * OW + 2), lambda c, j: (0, 0)),
            pl.BlockSpec((2 * row_blk, H), lambda c, j: (0, 0)),
        ],
        out_specs=(pl.BlockSpec((1, C * 4, j_rows, OW),
                                lambda c, j: (c * n_half + j, 0, 0, 0)),
                   pl.BlockSpec((1, C * 2, j_rows, 2),
                                lambda c, j: (c * n_half + j, 0, 0, 0))),
        compiler_params=cparams,
    )(x, rsel, lsel)
    xs = xs4.reshape(N, C * 4, lpad)

    A = ecol[..., 0]
    B = ecol[..., 1]
    E = jnp.concatenate([
        A[:, :, :OH], A[:, :, 1:OH + 1],
        B[:, :, 1:OH + 1], B[:, :, 2:OH + 2],
    ], axis=1)

    stats_kernel = functools.partial(_stats_kernel, m_out=m_out, ow=OW)
    acc = pl.pallas_call(
        stats_kernel,
        out_shape=jax.ShapeDtypeStruct((2, 2, OC, 128), f32),
        grid=(2, n_half),
        in_specs=[
            pl.BlockSpec((OC, 96), lambda c, j: (0, 0)),
            pl.BlockSpec((1, C * 4, lpad), lambda c, j: (c * n_half + j, 0, 0)),
            pl.BlockSpec((1, 32, OH), lambda c, j: (c * n_half + j, 0, 0)),
            pl.BlockSpec((OH, m_out), lambda c, j: (0, 0)),
        ],
        out_specs=pl.BlockSpec((1, 2, OC, 128), lambda c, j: (c, 0, 0, 0)),
        compiler_params=cparams,
    )(w_all, xs, E, spread)

    norm_kernel = functools.partial(
        _norm_kernel, m_rows=float(M), eps=eps, slope=negative_slope,
        m_out=m_out, ow=OW)
    out = pl.pallas_call(
        norm_kernel,
        out_shape=jax.ShapeDtypeStruct((N, OC, m_out), f32),
        grid=(2, n_half),
        in_specs=[
            pl.BlockSpec((OC, 96), lambda c, j: (0, 0)),
            pl.BlockSpec((2, 2, OC, 128), lambda c, j: (0, 0, 0, 0)),
            pl.BlockSpec((1, C * 4, lpad), lambda c, j: (c * n_half + j, 0, 0)),
            pl.BlockSpec((1, 32, OH), lambda c, j: (c * n_half + j, 0, 0)),
            pl.BlockSpec((OH, m_out), lambda c, j: (0, 0)),
        ],
        out_specs=pl.BlockSpec((1, OC, m_out),
                               lambda c, j: (c * n_half + j, 0, 0)),
        compiler_params=cparams,
    )(w_all, acc, xs, E, spread)

    return out.reshape(N, OC, OH, OW)


def kernel(x, weight):
    return _feature_layer(x, weight)


# speedup vs baseline: 25.5567x; 1.0500x over previous
import functools

import jax
import jax.numpy as jnp
import numpy as np
from jax.experimental import pallas as pl
from jax.experimental.pallas import tpu as pltpu


def _round_up(a, b):
    return (a + b - 1) // b * b


def _prep_kernel(x_ref, r_ref, l_ref, xs_ref, ecol_ref, *, c_in, row_blk,
                 j_rows, ow):
    for c in range(c_in):
        xb = x_ref[0, c].astype(jnp.bfloat16)
        t = jax.lax.dot_general(
            xb, r_ref[...],
            dimension_numbers=(((1,), (0,)), ((), ())),
            preferred_element_type=jnp.float32).astype(jnp.bfloat16)
        y = jax.lax.dot_general(
            l_ref[...], t,
            dimension_numbers=(((1,), (0,)), ((), ())),
            preferred_element_type=jnp.float32).astype(jnp.bfloat16)
        cols2 = jnp.concatenate([xb[:, -1:], xb[:, :1]], axis=1)
        e_t = jax.lax.dot_general(
            cols2, l_ref[...],
            dimension_numbers=(((0,), (1,)), ((), ())),
            preferred_element_type=jnp.float32).astype(jnp.bfloat16)
        for p in range(2):
            rows = y[p * row_blk:p * row_blk + j_rows]
            for q in range(2):
                xs_ref[0, c * 4 + p * 2 + q] = rows[:, q * ow:(q + 1) * ow]
            ecol_ref[0, c * 2 + p] = (
                e_t[:, p * row_blk:p * row_blk + j_rows])


def _conv_flat(w_ref, xs_ref, e_ref, spread_ref, m_out, ow, oh):
    taps = [xs_ref[0, :, off:off + m_out] for off in (0, 1, ow, ow + 1)]
    a_col = e_ref[0, :, 0, :]
    b_col = e_ref[0, :, 1, :]
    e_mat = jnp.concatenate([
        a_col[:, :oh], a_col[:, 1:oh + 1],
        b_col[:, 1:oh + 1], b_col[:, 2:oh + 2],
    ], axis=0)
    corr = jax.lax.dot_general(
        e_mat, spread_ref[...],
        dimension_numbers=(((1,), (0,)), ((), ())),
        preferred_element_type=jnp.float32).astype(jnp.bfloat16)
    rhs = jnp.concatenate(taps + [corr], axis=0)
    y = jax.lax.dot_general(
        w_ref[...], rhs,
        dimension_numbers=(((1,), (0,)), ((), ())),
        preferred_element_type=jnp.float32)
    return y


def _stats_kernel(w_ref, xs_ref, e_ref, spread_ref, acc_ref, *, m_out, ow, oh):
    j = pl.program_id(1)

    @pl.when(j == 0)
    def _():
        acc_ref[...] = jnp.zeros_like(acc_ref)

    y = _conv_flat(w_ref, xs_ref, e_ref, spread_ref, m_out, ow, oh)
    oc = acc_ref.shape[2]
    lanes = acc_ref.shape[3]
    s = jnp.sum(y, axis=1, keepdims=True)
    ss = jnp.sum(y * y, axis=1, keepdims=True)
    acc_ref[0, 0] += jnp.broadcast_to(s, (oc, lanes))
    acc_ref[0, 1] += jnp.broadcast_to(ss, (oc, lanes))


def _norm_kernel(w_ref, acc_ref, xs_ref, e_ref, spread_ref, out_ref, *,
                 m_rows, eps, slope, m_out, ow, oh):
    tot = acc_ref[0] + acc_ref[1]
    s = tot[0, :, :1]
    ss = tot[1, :, :1]
    mean = s * (1.0 / m_rows)
    denom_n = max(m_rows - 1.0, 1.0)
    var = jnp.maximum(ss - m_rows * mean * mean, 0.0) * (1.0 / denom_n)
    scale = 1.0 / (jnp.sqrt(var) + eps)

    y = _conv_flat(w_ref, xs_ref, e_ref, spread_ref, m_out, ow, oh)
    h = (y - mean) * scale
    out_ref[0] = jnp.maximum(h, slope * h)


def _feature_layer(x, weight, *, negative_slope=0.2, eps=1e-6):
    N, C, H, W = x.shape
    OC, Cw, KH, KW = weight.shape
    assert Cw == C and KH == 4 and KW == 4
    OH, OW = H // 2, W // 2
    M = N * OH * OW
    m_out = OH * OW
    j_rows = OH + 2
    row_blk = _round_up(j_rows, 16)
    lpad = j_rows * OW

    f32 = jnp.float32
    bf16 = jnp.bfloat16

    rsel_np = np.zeros((W, 2 * OW), np.float32)
    for q in range(2):
        for i in range(OW):
            w_src = 2 * i + q - 1
            if 0 <= w_src < W:
                rsel_np[w_src, q * OW + i] = 1.0
    rsel = jnp.asarray(rsel_np).astype(bf16)
    lsel_np = np.zeros((2 * row_blk, H), np.float32)
    for p in range(2):
        for j in range(j_rows):
            r_src = 2 * j + p - 1
            if 0 <= r_src < H:
                lsel_np[p * row_blk + j, r_src] = 1.0
    lsel = jnp.asarray(lsel_np).astype(bf16)

    n_half = N // 2
    cparams = pltpu.CompilerParams(
        dimension_semantics=("parallel", "arbitrary"),
        vmem_limit_bytes=64 * 1024 * 1024,
    )

    wr = weight.reshape(OC, C, 2, 2, 2, 2).transpose(2, 4, 0, 1, 3, 5)
    w_main = wr.reshape(4, OC, C * 4)
    w_main = jnp.concatenate([w_main[i] for i in range(4)], axis=1)
    wk = weight.reshape(OC, C, 2, 2, 4)
    wA = wk[..., 2].transpose(0, 2, 1, 3).reshape(OC, 4 * C)
    wB = (-wk[..., 3]).transpose(0, 2, 1, 3).reshape(OC, 4 * C)
    w_all = jnp.concatenate([w_main, wA, wB], axis=1).astype(bf16)

    spread_np = np.zeros((OH, m_out), np.float32)
    spread_np[np.arange(OH), np.arange(OH) * OW + (OW - 1)] = 1.0
    spread = jnp.asarray(spread_np).astype(bf16)

    prep_kernel = functools.partial(
        _prep_kernel, c_in=C, row_blk=row_blk, j_rows=j_rows, ow=OW)
    xs4, ecol = pl.pallas_call(
        prep_kernel,
        out_shape=(jax.ShapeDtypeStruct((N, C * 4, j_rows, OW), bf16),
                   jax.ShapeDtypeStruct((N, C * 2, 2, j_rows), bf16)),
        grid=(2, n_half),
        in_specs=[
            pl.BlockSpec((1, C, H, W), lambda c, j: (c * n_half + j, 0, 0, 0)),
            pl.BlockSpec((W, 2 * OW), lambda c, j: (0, 0)),
            pl.BlockSpec((2 * row_blk, H), lambda c, j: (0, 0)),
        ],
        out_specs=(pl.BlockSpec((1, C * 4, j_rows, OW),
                                lambda c, j: (c * n_half + j, 0, 0, 0)),
                   pl.BlockSpec((1, C * 2, 2, j_rows),
                                lambda c, j: (c * n_half + j, 0, 0, 0))),
        compiler_params=cparams,
    )(x, rsel, lsel)
    xs = xs4.reshape(N, C * 4, lpad)

    stats_kernel = functools.partial(_stats_kernel, m_out=m_out, ow=OW, oh=OH)
    acc = pl.pallas_call(
        stats_kernel,
        out_shape=jax.ShapeDtypeStruct((2, 2, OC, 128), f32),
        grid=(2, n_half),
        in_specs=[
            pl.BlockSpec((OC, 96), lambda c, j: (0, 0)),
            pl.BlockSpec((1, C * 4, lpad), lambda c, j: (c * n_half + j, 0, 0)),
            pl.BlockSpec((1, C * 2, 2, j_rows),
                         lambda c, j: (c * n_half + j, 0, 0, 0)),
            pl.BlockSpec((OH, m_out), lambda c, j: (0, 0)),
        ],
        out_specs=pl.BlockSpec((1, 2, OC, 128), lambda c, j: (c, 0, 0, 0)),
        compiler_params=cparams,
    )(w_all, xs, ecol, spread)

    norm_kernel = functools.partial(
        _norm_kernel, m_rows=float(M), eps=eps, slope=negative_slope,
        m_out=m_out, ow=OW, oh=OH)
    out = pl.pallas_call(
        norm_kernel,
        out_shape=jax.ShapeDtypeStruct((N, OC, m_out), f32),
        grid=(2, n_half),
        in_specs=[
            pl.BlockSpec((OC, 96), lambda c, j: (0, 0)),
            pl.BlockSpec((2, 2, OC, 128), lambda c, j: (0, 0, 0, 0)),
            pl.BlockSpec((1, C * 4, lpad), lambda c, j: (c * n_half + j, 0, 0)),
            pl.BlockSpec((1, C * 2, 2, j_rows),
                         lambda c, j: (c * n_half + j, 0, 0, 0)),
            pl.BlockSpec((OH, m_out), lambda c, j: (0, 0)),
        ],
        out_specs=pl.BlockSpec((1, OC, m_out),
                               lambda c, j: (c * n_half + j, 0, 0)),
        compiler_params=cparams,
    )(w_all, acc, xs, ecol, spread)

    return out.reshape(N, OC, OH, OW)


def kernel(x, weight):
    return _feature_layer(x, weight)
